# baseline (speedup 1.0000x reference)
import functools
import jax
import jax.numpy as jnp
from jax.experimental import pallas as pl
from jax.experimental.pallas import tpu as pltpu
from jax.experimental.pallas import tpu_sc as plsc

_LANE = 128
_SCL = 16
_RB = 4


def _sc_cumsum_body(in_vmem, out_vmem):
    n = in_vmem.shape[1]

    def chunk(c, carries):
        new = []
        for r in range(_RB):
            v = in_vmem[r, pl.ds(c * _SCL, _SCL)]
            y = plsc.cumsum(v) + carries[r]
            out_vmem[r, pl.ds(c * _SCL, _SCL)] = y
            new.append(carries[r] + jnp.sum(v))
        return tuple(new)

    jax.lax.fori_loop(0, n // _SCL, chunk,
                      tuple(jnp.float32(0.0) for _ in range(_RB)))


def _sc_cumsum_rows(x2d):
    rows, n = x2d.shape
    mesh = plsc.VectorSubcoreMesh(core_axis_name="c", subcore_axis_name="s")

    @functools.partial(
        pl.kernel, mesh=mesh,
        out_type=jax.ShapeDtypeStruct((rows, n), jnp.float32),
        compiler_params=pltpu.CompilerParams(needs_layout_passes=False))
    def k(x_hbm, o_hbm):
        pltpu.emit_pipeline(
            _sc_cumsum_body,
            grid=(rows // _RB,),
            in_specs=[pl.BlockSpec((_RB, n), lambda i: (i, 0))],
            out_specs=[pl.BlockSpec((_RB, n), lambda i: (i, 0))],
            core_axis_name=("c", "s"),
            dimension_semantics=(pltpu.PARALLEL,),
        )(x_hbm, o_hbm)

    return k(x2d)


def _cumsum_kernel(x_ref, o_ref):
    n = x_ref.shape[1]
    chunks = n // _LANE

    ri = jax.lax.broadcasted_iota(jnp.int32, (_LANE, _LANE), 0)
    ci = jax.lax.broadcasted_iota(jnp.int32, (_LANE, _LANE), 1)
    tri = (ri <= ci).astype(jnp.bfloat16)

    dims = (((1,), (0,)), ((), ()))
    carry = jnp.zeros((x_ref.shape[0], 1), jnp.float32)
    for c in range(chunks):
        xc = x_ref[:, c * _LANE:(c + 1) * _LANE]
        hi = xc.astype(jnp.bfloat16)
        lo = (xc - hi.astype(jnp.float32)).astype(jnp.bfloat16)
        yc = jax.lax.dot_general(
            hi, tri, dims, preferred_element_type=jnp.float32)
        yc = yc + jax.lax.dot_general(
            lo, tri, dims, preferred_element_type=jnp.float32)
        o_ref[:, c * _LANE:(c + 1) * _LANE] = yc + carry
        carry = carry + yc[:, _LANE - 1:_LANE]


def _cumsum_rows(x2d, block_rows, interpret=False):
    rows, n = x2d.shape
    grid = (rows // block_rows,)
    return pl.pallas_call(
        _cumsum_kernel,
        grid=grid,
        in_specs=[pl.BlockSpec((block_rows, n), lambda i: (i, 0))],
        out_specs=pl.BlockSpec((block_rows, n), lambda i: (i, 0)),
        out_shape=jax.ShapeDtypeStruct((rows, n), jnp.float32),
        interpret=interpret,
    )(x2d)


def kernel(x):
    b, s, n = x.shape
    x2d = x.reshape(b * s, n).astype(jnp.float32)
    out = _sc_cumsum_rows(x2d)
    return out.reshape(b, s, n).astype(x.dtype)


# speedup vs baseline: 8.9887x; 8.9887x over previous
import functools
import jax
import jax.numpy as jnp
from jax.experimental import pallas as pl
from jax.experimental.pallas import tpu as pltpu
from jax.experimental.pallas import tpu_sc as plsc

_LANE = 128
_SCL = 16
_RB = 4


def _sc_cumsum_body(in_vmem, out_vmem):
    n = in_vmem.shape[1]

    def chunk(c, carries):
        new = []
        for r in range(_RB):
            v = in_vmem[r, pl.ds(c * _SCL, _SCL)]
            y = plsc.cumsum(v) + carries[r]
            out_vmem[r, pl.ds(c * _SCL, _SCL)] = y
            new.append(carries[r] + jnp.sum(v))
        return tuple(new)

    jax.lax.fori_loop(0, n // _SCL, chunk,
                      tuple(jnp.float32(0.0) for _ in range(_RB)))


def _sc_cumsum_rows(x2d):
    rows, n = x2d.shape
    mesh = plsc.VectorSubcoreMesh(core_axis_name="c", subcore_axis_name="s")

    @functools.partial(
        pl.kernel, mesh=mesh,
        out_type=jax.ShapeDtypeStruct((rows, n), jnp.float32),
        compiler_params=pltpu.CompilerParams(needs_layout_passes=False))
    def k(x_hbm, o_hbm):
        pltpu.emit_pipeline(
            _sc_cumsum_body,
            grid=(rows // _RB,),
            in_specs=[pl.BlockSpec((_RB, n), lambda i: (i, 0))],
            out_specs=[pl.BlockSpec((_RB, n), lambda i: (i, 0))],
            core_axis_name=("c", "s"),
            dimension_semantics=(pltpu.PARALLEL,),
        )(x_hbm, o_hbm)

    return k(x2d)


def _cumsum_kernel(x_ref, o_ref):
    n = x_ref.shape[1]
    chunks = n // _LANE

    ri = jax.lax.broadcasted_iota(jnp.int32, (_LANE, _LANE), 0)
    ci = jax.lax.broadcasted_iota(jnp.int32, (_LANE, _LANE), 1)
    tri = (ri <= ci).astype(jnp.bfloat16)

    dims = (((1,), (0,)), ((), ()))
    carry = jnp.zeros((x_ref.shape[0], 1), jnp.float32)
    for c in range(chunks):
        xc = x_ref[:, c * _LANE:(c + 1) * _LANE]
        hi = xc.astype(jnp.bfloat16)
        lo = (xc - hi.astype(jnp.float32)).astype(jnp.bfloat16)
        yc = jax.lax.dot_general(
            hi, tri, dims, preferred_element_type=jnp.float32)
        yc = yc + jax.lax.dot_general(
            lo, tri, dims, preferred_element_type=jnp.float32)
        o_ref[:, c * _LANE:(c + 1) * _LANE] = yc + carry
        carry = carry + yc[:, _LANE - 1:_LANE]


def _cumsum_rows(x2d, block_rows, interpret=False):
    rows, n = x2d.shape
    grid = (pl.cdiv(rows, block_rows),)
    return pl.pallas_call(
        _cumsum_kernel,
        grid=grid,
        in_specs=[pl.BlockSpec((block_rows, n), lambda i: (i, 0))],
        out_specs=pl.BlockSpec((block_rows, n), lambda i: (i, 0)),
        out_shape=jax.ShapeDtypeStruct((rows, n), jnp.float32),
        compiler_params=pltpu.CompilerParams(
            vmem_limit_bytes=110 * 1024 * 1024),
        interpret=interpret,
    )(x2d)


def kernel(x):
    b, s, n = x.shape
    x2d = x.reshape(b * s, n).astype(jnp.float32)
    out = _cumsum_rows(x2d, block_rows=768)
    return out.reshape(b, s, n).astype(x.dtype)
